# baseline (speedup 1.0000x reference)
import functools

import jax
import jax.numpy as jnp
from jax import lax
from jax.experimental import pallas as pl
from jax.experimental.pallas import tpu as pltpu
from jax.experimental.pallas import tpu_sc as plsc

N, D, H, E, TOPK = 2048, 768, 3072, 8, 2
TILE = 256
NT = (N * TOPK) // TILE + E
NPAD = NT * TILE
NC, NS, NW = 2, 16, 32
NEG_INF = float("-inf")


def _router_body(x_ref, w2_ref, znt_ref, pos_ref, gate_ref, te_ref):
    lg = lax.dot_general(w2_ref[...], x_ref[...], (((1,), (1,)), ((), ())),
                         preferred_element_type=jnp.float32,
                         precision=lax.Precision.DEFAULT)
    logits = lg[:E]
    nlog = lg[E:]
    softplus = jnp.maximum(nlog, 0.0) + jnp.log(1.0 + jnp.exp(-jnp.abs(nlog)))
    noisy = logits + znt_ref[...] * softplus

    iota_e = lax.broadcasted_iota(jnp.int32, (E, N), 0)
    v0 = jnp.max(noisy, axis=0, keepdims=True)
    i0 = jnp.min(jnp.where(noisy == v0, iota_e, E), axis=0, keepdims=True)
    sel0 = iota_e == i0
    noisy1 = jnp.where(sel0, NEG_INF, noisy)
    v1 = jnp.max(noisy1, axis=0, keepdims=True)
    i1 = jnp.min(jnp.where(noisy1 == v1, iota_e, E), axis=0, keepdims=True)
    sel1 = iota_e == i1
    r = jnp.exp(v1 - v0)
    g0 = 1.0 / (1.0 + r)
    g1 = r / (1.0 + r)

    m = (sel0 | sel1).astype(jnp.int32)
    iota_t = lax.broadcasted_iota(jnp.int32, (E, N), 1)
    c = m
    s = 1
    while s < N:
        c = c + jnp.where(iota_t >= s, pltpu.roll(c, s, 1), 0)
        s *= 2
    rank = c - m
    counts = c[:, N - 1:N]
    sizes = ((counts + TILE - 1) // TILE) * TILE
    iota_e1 = lax.broadcasted_iota(jnp.int32, (E, 1), 0)
    oc = sizes
    s = 1
    while s < E:
        oc = oc + jnp.where(iota_e1 >= s, pltpu.roll(oc, s, 0), 0)
        s *= 2
    off = oc - sizes

    base = off + rank
    pos0 = jnp.sum(jnp.where(sel0, base, 0), axis=0, keepdims=True)
    pos1 = jnp.sum(jnp.where(sel1, base, 0), axis=0, keepdims=True)
    pos_ref[...] = jnp.concatenate([pos0, pos1], axis=0)
    gate_ref[...] = jnp.concatenate([g0, g1], axis=0)

    tiles_before = off // TILE
    total_tiles = (off[E - 1:E, :] + sizes[E - 1:E, :]) // TILE
    iota_nt = lax.broadcasted_iota(jnp.int32, (E, NT), 1)
    cnt = jnp.sum((tiles_before <= iota_nt).astype(jnp.int32), axis=0,
                  keepdims=True) - 1
    iota_row = lax.broadcasted_iota(jnp.int32, (1, NT), 1)
    te_ref[...] = jnp.where(iota_row < total_tiles, cnt, -1)


def _run_router(flat_x, w2, znt):
    return pl.pallas_call(
        _router_body,
        out_shape=[
            jax.ShapeDtypeStruct((TOPK, N), jnp.int32),
            jax.ShapeDtypeStruct((TOPK, N), jnp.float32),
            jax.ShapeDtypeStruct((1, NT), jnp.int32),
        ],
    )(flat_x, w2, znt)


def _scatter_body(pos_hbm, w_hbm, tok_out, gate_out, pos_v, w_v, dtok_v,
                  dgate_v):
    first = (lax.axis_index("c") == 0) & (lax.axis_index("s") == 0)

    @pl.when(first)
    def _():
        pltpu.sync_copy(pos_hbm, pos_v)
        pltpu.sync_copy(w_hbm, w_v)
        zero_i = jnp.zeros((16,), jnp.int32)
        zero_f = jnp.zeros((16,), jnp.float32)

        def zero_step(q, _):
            dtok_v[pl.ds(q * 16, 16)] = zero_i
            dgate_v[pl.ds(q * 16, 16)] = zero_f
            return 0

        lax.fori_loop(0, NPAD // 16, zero_step, 0)
        lane = lax.iota(jnp.int32, 16)

        def scat_step(q, _):
            idx = pos_v[pl.ds(q * 16, 16)]
            tok = (lane + q * 16) >> 1
            plsc.store_scatter(dtok_v, [idx], tok)
            plsc.store_scatter(dgate_v, [idx], w_v[pl.ds(q * 16, 16)])
            return 0

        lax.fori_loop(0, (N * TOPK) // 16, scat_step, 0)
        pltpu.sync_copy(dtok_v, tok_out)
        pltpu.sync_copy(dgate_v, gate_out)


def _run_scatter(pos_flat, w_flat):
    mesh = plsc.VectorSubcoreMesh(core_axis_name="c", subcore_axis_name="s")
    f = pl.kernel(
        _scatter_body,
        out_type=[
            jax.ShapeDtypeStruct((NPAD,), jnp.int32),
            jax.ShapeDtypeStruct((NPAD,), jnp.float32),
        ],
        mesh=mesh,
        compiler_params=pltpu.CompilerParams(needs_layout_passes=False),
        scratch_types=[
            pltpu.VMEM((N * TOPK,), jnp.int32),
            pltpu.VMEM((N * TOPK,), jnp.float32),
            pltpu.VMEM((NPAD,), jnp.int32),
            pltpu.VMEM((NPAD,), jnp.float32),
        ],
    )
    return f(pos_flat, w_flat)


_GROWS = NPAD // NW
_GCHUNK = _GROWS // 4


_GC = _GROWS // 3


def _gather_body(tok_hbm, x_hbm, out_hbm, idx_v, rows0, rows1, gsem, wsem):
    wid = lax.axis_index("s") * NC + lax.axis_index("c")
    base = wid * _GROWS
    pltpu.sync_copy(tok_hbm.at[pl.ds(base, _GROWS)], idx_v)
    bufs = (rows0, rows1, rows0)
    g = [None, None, None]
    w = [None, None, None]
    g[0] = pltpu.async_copy(x_hbm.at[idx_v.at[pl.ds(0, _GC)]], bufs[0], gsem)
    g[1] = pltpu.async_copy(x_hbm.at[idx_v.at[pl.ds(_GC, _GC)]], bufs[1], gsem)
    g[0].wait()
    w[0] = pltpu.async_copy(bufs[0], out_hbm.at[pl.ds(base, _GC)], wsem)
    g[1].wait()
    w[1] = pltpu.async_copy(bufs[1], out_hbm.at[pl.ds(base + _GC, _GC)], wsem)
    w[0].wait()
    g[2] = pltpu.async_copy(
        x_hbm.at[idx_v.at[pl.ds(2 * _GC, _GC)]], bufs[2], gsem)
    g[2].wait()
    w[2] = pltpu.async_copy(
        bufs[2], out_hbm.at[pl.ds(base + 2 * _GC, _GC)], wsem)
    w[1].wait()
    w[2].wait()


def _run_gather(row_token, flat_x):
    mesh = plsc.VectorSubcoreMesh(core_axis_name="c", subcore_axis_name="s")
    f = pl.kernel(
        _gather_body,
        out_type=jax.ShapeDtypeStruct((NPAD, D), jnp.float32),
        mesh=mesh,
        compiler_params=pltpu.CompilerParams(needs_layout_passes=False),
        scratch_types=[
            pltpu.VMEM((_GROWS,), jnp.int32),
            pltpu.VMEM((_GC, D), jnp.float32),
            pltpu.VMEM((_GC, D), jnp.float32),
            pltpu.SemaphoreType.DMA,
            pltpu.SemaphoreType.DMA,
        ],
    )
    return f(row_token, flat_x)


def _ffn_body(te_ref, tok_ref, x_ref, fc1_ref, fc2_ref, g_ref, out_ref):
    i = pl.program_id(0)
    active = te_ref[i] >= 0

    @pl.when(active)
    def _():
        tok = tok_ref[...]
        iota_n = lax.broadcasted_iota(jnp.int32, (TILE, N), 1)
        p = (iota_n == tok).astype(jnp.float32)
        xb = lax.dot_general(p, x_ref[...], (((1,), (0,)), ((), ())),
                             preferred_element_type=jnp.float32)
        h = lax.dot_general(xb, fc1_ref[0], (((1,), (1,)), ((), ())),
                            preferred_element_type=jnp.float32)
        h = jnp.maximum(h, 0.0)
        h = h * h
        y = lax.dot_general(h, fc2_ref[0], (((1,), (1,)), ((), ())),
                            preferred_element_type=jnp.float32)
        out_ref[...] = y * g_ref[...]

    @pl.when(jnp.logical_not(active))
    def _():
        out_ref[...] = jnp.zeros_like(out_ref)


def _run_ffn(te, row_token, xb, fc1_W, fc2_W, row_gate):
    grid_spec = pltpu.PrefetchScalarGridSpec(
        num_scalar_prefetch=1,
        grid=(NT,),
        in_specs=[
            pl.BlockSpec((TILE, 1), lambda i, te: (i, 0)),
            pl.BlockSpec((N, D), lambda i, te: (0, 0)),
            pl.BlockSpec((1, H, D), lambda i, te: (jnp.maximum(te[i], 0), 0, 0)),
            pl.BlockSpec((1, D, H), lambda i, te: (jnp.maximum(te[i], 0), 0, 0)),
            pl.BlockSpec((TILE, 1), lambda i, te: (i, 0)),
        ],
        out_specs=pl.BlockSpec((TILE, D), lambda i, te: (i, 0)),
    )
    return pl.pallas_call(
        _ffn_body,
        grid_spec=grid_spec,
        out_shape=jax.ShapeDtypeStruct((NPAD, D), jnp.float32),
        compiler_params=pltpu.CompilerParams(
            dimension_semantics=("arbitrary",)),
    )(te, row_token, xb, fc1_W, fc2_W, row_gate)


_TCHUNK = N // NW // 2


def _combine_body(pos_hbm, y_hbm, out_hbm, idx_v, rows_v, out_v, sem):
    wid = lax.axis_index("s") * NC + lax.axis_index("c")
    for c in range(2):
        tbase = wid * (2 * _TCHUNK) + c * _TCHUNK
        pltpu.sync_copy(pos_hbm.at[pl.ds(2 * tbase, 2 * _TCHUNK)], idx_v)
        pltpu.async_copy(y_hbm.at[idx_v], rows_v, sem).wait()

        def add_step(j, _):
            for k in range(D // 16):
                out_v[j, pl.ds(k * 16, 16)] = (
                    rows_v[2 * j, pl.ds(k * 16, 16)]
                    + rows_v[2 * j + 1, pl.ds(k * 16, 16)])
            return 0

        lax.fori_loop(0, _TCHUNK, add_step, 0)
        pltpu.sync_copy(out_v, out_hbm.at[pl.ds(tbase, _TCHUNK)])


def _run_combine(pos_flat, y):
    mesh = plsc.VectorSubcoreMesh(core_axis_name="c", subcore_axis_name="s")
    f = pl.kernel(
        _combine_body,
        out_type=jax.ShapeDtypeStruct((N, D), jnp.float32),
        mesh=mesh,
        compiler_params=pltpu.CompilerParams(needs_layout_passes=False),
        scratch_types=[
            pltpu.VMEM((2 * _TCHUNK,), jnp.int32),
            pltpu.VMEM((2 * _TCHUNK, D), jnp.float32),
            pltpu.VMEM((_TCHUNK, D), jnp.float32),
            pltpu.SemaphoreType.DMA,
        ],
    )
    return f(pos_flat, y)


def kernel(x, route_W, noise_W, fc1_W, fc2_W):
    b, t, d = x.shape
    flat_x = x.reshape(-1, d)
    w2 = jnp.concatenate([route_W, noise_W], axis=0)
    znt = jax.random.normal(jax.random.key(1234), (N, E), jnp.float32).T

    pos, gates, te = _run_router(flat_x, w2, znt)
    pos_flat = pos.T.reshape(-1)
    w_flat = gates.T.reshape(-1)
    te_flat = te.reshape(-1)

    row_token, row_gate = _run_scatter(pos_flat, w_flat)
    y = _run_ffn(te_flat, row_token.reshape(NPAD, 1), flat_x, fc1_W, fc2_W,
                 row_gate.reshape(NPAD, 1))
    out = _run_combine(pos_flat, y)
    return out.reshape(b, t, d)


# speedup vs baseline: 1.0718x; 1.0718x over previous
import functools

import jax
import jax.numpy as jnp
from jax import lax
from jax.experimental import pallas as pl
from jax.experimental.pallas import tpu as pltpu
from jax.experimental.pallas import tpu_sc as plsc

N, D, H, E, TOPK = 2048, 768, 3072, 8, 2
TILE = 256
NT = (N * TOPK) // TILE + E
NPAD = NT * TILE
NC, NS, NW = 2, 16, 32
NEG_INF = float("-inf")


def _router_body(x_ref, w2_ref, znt_ref, pos_ref, gate_ref, te_ref):
    lg = lax.dot_general(w2_ref[...], x_ref[...], (((1,), (1,)), ((), ())),
                         preferred_element_type=jnp.float32,
                         precision=lax.Precision.DEFAULT)
    logits = lg[:E]
    nlog = lg[E:]
    softplus = jnp.maximum(nlog, 0.0) + jnp.log(1.0 + jnp.exp(-jnp.abs(nlog)))
    noisy = logits + znt_ref[...] * softplus

    iota_e = lax.broadcasted_iota(jnp.int32, (E, N), 0)
    v0 = jnp.max(noisy, axis=0, keepdims=True)
    i0 = jnp.min(jnp.where(noisy == v0, iota_e, E), axis=0, keepdims=True)
    sel0 = iota_e == i0
    noisy1 = jnp.where(sel0, NEG_INF, noisy)
    v1 = jnp.max(noisy1, axis=0, keepdims=True)
    i1 = jnp.min(jnp.where(noisy1 == v1, iota_e, E), axis=0, keepdims=True)
    sel1 = iota_e == i1
    r = jnp.exp(v1 - v0)
    g0 = 1.0 / (1.0 + r)
    g1 = r / (1.0 + r)

    m = (sel0 | sel1).astype(jnp.int32)
    iota_t = lax.broadcasted_iota(jnp.int32, (E, N), 1)
    c = m
    s = 1
    while s < N:
        c = c + jnp.where(iota_t >= s, pltpu.roll(c, s, 1), 0)
        s *= 2
    rank = c - m
    counts = c[:, N - 1:N]
    sizes = ((counts + TILE - 1) // TILE) * TILE
    iota_e1 = lax.broadcasted_iota(jnp.int32, (E, 1), 0)
    oc = sizes
    s = 1
    while s < E:
        oc = oc + jnp.where(iota_e1 >= s, pltpu.roll(oc, s, 0), 0)
        s *= 2
    off = oc - sizes

    base = off + rank
    pos0 = jnp.sum(jnp.where(sel0, base, 0), axis=0, keepdims=True)
    pos1 = jnp.sum(jnp.where(sel1, base, 0), axis=0, keepdims=True)
    pos_ref[...] = jnp.concatenate([pos0, pos1], axis=0)
    gate_ref[...] = jnp.concatenate([g0, g1], axis=0)

    tiles_before = off // TILE
    total_tiles = (off[E - 1:E, :] + sizes[E - 1:E, :]) // TILE
    iota_nt = lax.broadcasted_iota(jnp.int32, (E, NT), 1)
    cnt = jnp.sum((tiles_before <= iota_nt).astype(jnp.int32), axis=0,
                  keepdims=True) - 1
    iota_row = lax.broadcasted_iota(jnp.int32, (1, NT), 1)
    te_ref[...] = jnp.where(iota_row < total_tiles, cnt, -1)


def _run_router(flat_x, w2, znt):
    return pl.pallas_call(
        _router_body,
        out_shape=[
            jax.ShapeDtypeStruct((TOPK, N), jnp.int32),
            jax.ShapeDtypeStruct((TOPK, N), jnp.float32),
            jax.ShapeDtypeStruct((1, NT), jnp.int32),
        ],
    )(flat_x, w2, znt)


_SROWS = NPAD // NW


def _scatter_body(pos_hbm, w_hbm, tok_out, gate_out, pos_v, w_v, dtok_v,
                  dgate_v):
    wid = lax.axis_index("s") * NC + lax.axis_index("c")
    base = wid * _SROWS
    pltpu.sync_copy(pos_hbm, pos_v)
    pltpu.sync_copy(w_hbm, w_v)
    lane = lax.iota(jnp.int32, 16)

    def scat_step(q, _):
        idx = pos_v[pl.ds(q * 16, 16)] - base
        mask = (idx >= 0) & (idx < _SROWS)
        tok = (lane + q * 16) & (N - 1)
        plsc.store_scatter(dtok_v, [idx], tok, mask=mask)
        plsc.store_scatter(dgate_v, [idx], w_v[pl.ds(q * 16, 16)], mask=mask)
        return 0

    lax.fori_loop(0, (N * TOPK) // 16, scat_step, 0)
    pltpu.sync_copy(dtok_v, tok_out.at[pl.ds(base, _SROWS)])
    pltpu.sync_copy(dgate_v, gate_out.at[pl.ds(base, _SROWS)])


def _run_scatter(pos_flat, w_flat):
    mesh = plsc.VectorSubcoreMesh(core_axis_name="c", subcore_axis_name="s")
    f = pl.kernel(
        _scatter_body,
        out_type=[
            jax.ShapeDtypeStruct((NPAD,), jnp.int32),
            jax.ShapeDtypeStruct((NPAD,), jnp.float32),
        ],
        mesh=mesh,
        compiler_params=pltpu.CompilerParams(needs_layout_passes=False),
        scratch_types=[
            pltpu.VMEM((N * TOPK,), jnp.int32),
            pltpu.VMEM((N * TOPK,), jnp.float32),
            pltpu.VMEM((_SROWS,), jnp.int32),
            pltpu.VMEM((_SROWS,), jnp.float32),
        ],
    )
    return f(pos_flat, w_flat)


_GROWS = NPAD // NW
_GCHUNK = _GROWS // 4


_GC = _GROWS // 3


def _gather_body(tok_hbm, x_hbm, out_hbm, idx_v, rows0, rows1, gsem, wsem):
    wid = lax.axis_index("s") * NC + lax.axis_index("c")
    base = wid * _GROWS
    pltpu.sync_copy(tok_hbm.at[pl.ds(base, _GROWS)], idx_v)
    bufs = (rows0, rows1, rows0)
    g = [None, None, None]
    w = [None, None, None]
    g[0] = pltpu.async_copy(x_hbm.at[idx_v.at[pl.ds(0, _GC)]], bufs[0], gsem)
    g[1] = pltpu.async_copy(x_hbm.at[idx_v.at[pl.ds(_GC, _GC)]], bufs[1], gsem)
    g[0].wait()
    w[0] = pltpu.async_copy(bufs[0], out_hbm.at[pl.ds(base, _GC)], wsem)
    g[1].wait()
    w[1] = pltpu.async_copy(bufs[1], out_hbm.at[pl.ds(base + _GC, _GC)], wsem)
    w[0].wait()
    g[2] = pltpu.async_copy(
        x_hbm.at[idx_v.at[pl.ds(2 * _GC, _GC)]], bufs[2], gsem)
    g[2].wait()
    w[2] = pltpu.async_copy(
        bufs[2], out_hbm.at[pl.ds(base + 2 * _GC, _GC)], wsem)
    w[1].wait()
    w[2].wait()


def _run_gather(row_token, flat_x):
    mesh = plsc.VectorSubcoreMesh(core_axis_name="c", subcore_axis_name="s")
    f = pl.kernel(
        _gather_body,
        out_type=jax.ShapeDtypeStruct((NPAD, D), jnp.float32),
        mesh=mesh,
        compiler_params=pltpu.CompilerParams(needs_layout_passes=False),
        scratch_types=[
            pltpu.VMEM((_GROWS,), jnp.int32),
            pltpu.VMEM((_GC, D), jnp.float32),
            pltpu.VMEM((_GC, D), jnp.float32),
            pltpu.SemaphoreType.DMA,
            pltpu.SemaphoreType.DMA,
        ],
    )
    return f(row_token, flat_x)


def _ffn_body(te_ref, tok_ref, x_ref, fc1_ref, fc2_ref, g_ref, out_ref):
    i = pl.program_id(0)
    active = te_ref[i] >= 0

    @pl.when(active)
    def _():
        tok = tok_ref[...]
        iota_n = lax.broadcasted_iota(jnp.int32, (TILE, N), 1)
        p = (iota_n == tok).astype(jnp.float32)
        xb = lax.dot_general(p, x_ref[...], (((1,), (0,)), ((), ())),
                             preferred_element_type=jnp.float32)
        h = lax.dot_general(xb, fc1_ref[0], (((1,), (1,)), ((), ())),
                            preferred_element_type=jnp.float32)
        h = jnp.maximum(h, 0.0)
        h = h * h
        y = lax.dot_general(h, fc2_ref[0], (((1,), (1,)), ((), ())),
                            preferred_element_type=jnp.float32)
        out_ref[...] = y * g_ref[...]

    @pl.when(jnp.logical_not(active))
    def _():
        out_ref[...] = jnp.zeros_like(out_ref)


def _run_ffn(te, row_token, xb, fc1_W, fc2_W, row_gate):
    grid_spec = pltpu.PrefetchScalarGridSpec(
        num_scalar_prefetch=1,
        grid=(NT,),
        in_specs=[
            pl.BlockSpec((TILE, 1), lambda i, te: (i, 0)),
            pl.BlockSpec((N, D), lambda i, te: (0, 0)),
            pl.BlockSpec((1, H, D), lambda i, te: (jnp.maximum(te[i], 0), 0, 0)),
            pl.BlockSpec((1, D, H), lambda i, te: (jnp.maximum(te[i], 0), 0, 0)),
            pl.BlockSpec((TILE, 1), lambda i, te: (i, 0)),
        ],
        out_specs=pl.BlockSpec((TILE, D), lambda i, te: (i, 0)),
    )
    return pl.pallas_call(
        _ffn_body,
        grid_spec=grid_spec,
        out_shape=jax.ShapeDtypeStruct((NPAD, D), jnp.float32),
        compiler_params=pltpu.CompilerParams(
            dimension_semantics=("arbitrary",)),
    )(te, row_token, xb, fc1_W, fc2_W, row_gate)


_TOKW = N // NW
_TCHUNK = _TOKW // 2


def _combine_body(pos_hbm, y_hbm, out_hbm, idx0_v, idx1_v, r0, r1, oa, ob,
                  gs0, gs1, wsem):
    wid = lax.axis_index("s") * NC + lax.axis_index("c")
    tw = wid * _TOKW
    pltpu.sync_copy(pos_hbm.at[pl.ds(tw, _TOKW)], idx0_v)
    pltpu.sync_copy(pos_hbm.at[pl.ds(N + tw, _TOKW)], idx1_v)
    outs = (oa, ob)
    wd = [None, None]
    for c in range(2):
        tb = tw + c * _TCHUNK
        g0 = pltpu.async_copy(
            y_hbm.at[idx0_v.at[pl.ds(c * _TCHUNK, _TCHUNK)]], r0, gs0)
        g1 = pltpu.async_copy(
            y_hbm.at[idx1_v.at[pl.ds(c * _TCHUNK, _TCHUNK)]], r1, gs1)
        g0.wait()
        g1.wait()
        o = outs[c]

        def add_step(j, _):
            for k in range(D // 16):
                o[j, pl.ds(k * 16, 16)] = (
                    r0[j, pl.ds(k * 16, 16)] + r1[j, pl.ds(k * 16, 16)])
            return 0

        lax.fori_loop(0, _TCHUNK, add_step, 0)
        wd[c] = pltpu.async_copy(o, out_hbm.at[pl.ds(tb, _TCHUNK)], wsem)
    wd[0].wait()
    wd[1].wait()


def _run_combine(pos_flat, y):
    mesh = plsc.VectorSubcoreMesh(core_axis_name="c", subcore_axis_name="s")
    f = pl.kernel(
        _combine_body,
        out_type=jax.ShapeDtypeStruct((N, D), jnp.float32),
        mesh=mesh,
        compiler_params=pltpu.CompilerParams(needs_layout_passes=False),
        scratch_types=[
            pltpu.VMEM((_TOKW,), jnp.int32),
            pltpu.VMEM((_TOKW,), jnp.int32),
            pltpu.VMEM((_TCHUNK, D), jnp.float32),
            pltpu.VMEM((_TCHUNK, D), jnp.float32),
            pltpu.VMEM((_TCHUNK, D), jnp.float32),
            pltpu.VMEM((_TCHUNK, D), jnp.float32),
            pltpu.SemaphoreType.DMA,
            pltpu.SemaphoreType.DMA,
            pltpu.SemaphoreType.DMA,
        ],
    )
    return f(pos_flat, y)


def kernel(x, route_W, noise_W, fc1_W, fc2_W):
    b, t, d = x.shape
    flat_x = x.reshape(-1, d)
    w2 = jnp.concatenate([route_W, noise_W], axis=0)
    znt = jax.random.normal(jax.random.key(1234), (N, E), jnp.float32).T

    pos, gates, te = _run_router(flat_x, w2, znt)
    pos_flat = pos.reshape(-1)
    w_flat = gates.reshape(-1)
    te_flat = te.reshape(-1)

    row_token, row_gate = _run_scatter(pos_flat, w_flat)
    y = _run_ffn(te_flat, row_token.reshape(NPAD, 1), flat_x, fc1_W, fc2_W,
                 row_gate.reshape(NPAD, 1))
    out = _run_combine(pos_flat, y)
    return out.reshape(b, t, d)


# speedup vs baseline: 1.0828x; 1.0103x over previous
import functools

import jax
import jax.numpy as jnp
from jax import lax
from jax.experimental import pallas as pl
from jax.experimental.pallas import tpu as pltpu
from jax.experimental.pallas import tpu_sc as plsc

N, D, H, E, TOPK = 2048, 768, 3072, 8, 2
TILE = 256
NT = (N * TOPK) // TILE + E
NPAD = NT * TILE
NC, NS, NW = 2, 16, 32
NEG_INF = float("-inf")


def _router_body(x_ref, w2_ref, znt_ref, pos_ref, gate_ref, te_ref):
    lg = lax.dot_general(w2_ref[...], x_ref[...], (((1,), (1,)), ((), ())),
                         preferred_element_type=jnp.float32,
                         precision=lax.Precision.DEFAULT)
    logits = lg[:E]
    nlog = lg[E:]
    softplus = jnp.maximum(nlog, 0.0) + jnp.log(1.0 + jnp.exp(-jnp.abs(nlog)))
    noisy = logits + znt_ref[...] * softplus

    iota_e = lax.broadcasted_iota(jnp.int32, (E, N), 0)
    v0 = jnp.max(noisy, axis=0, keepdims=True)
    i0 = jnp.min(jnp.where(noisy == v0, iota_e, E), axis=0, keepdims=True)
    sel0 = iota_e == i0
    noisy1 = jnp.where(sel0, NEG_INF, noisy)
    v1 = jnp.max(noisy1, axis=0, keepdims=True)
    i1 = jnp.min(jnp.where(noisy1 == v1, iota_e, E), axis=0, keepdims=True)
    sel1 = iota_e == i1
    r = jnp.exp(v1 - v0)
    g0 = 1.0 / (1.0 + r)
    g1 = r / (1.0 + r)

    m = (sel0 | sel1).astype(jnp.int32)
    iota_t = lax.broadcasted_iota(jnp.int32, (E, N), 1)
    c = m
    s = 1
    while s < N:
        c = c + jnp.where(iota_t >= s, pltpu.roll(c, s, 1), 0)
        s *= 2
    rank = c - m
    counts = c[:, N - 1:N]
    sizes = ((counts + TILE - 1) // TILE) * TILE
    iota_e1 = lax.broadcasted_iota(jnp.int32, (E, 1), 0)
    oc = sizes
    s = 1
    while s < E:
        oc = oc + jnp.where(iota_e1 >= s, pltpu.roll(oc, s, 0), 0)
        s *= 2
    off = oc - sizes

    base = off + rank
    pos0 = jnp.sum(jnp.where(sel0, base, 0), axis=0, keepdims=True)
    pos1 = jnp.sum(jnp.where(sel1, base, 0), axis=0, keepdims=True)
    pos_ref[...] = jnp.concatenate([pos0, pos1], axis=0)
    gate_ref[...] = jnp.concatenate([g0, g1], axis=0)

    tiles_before = off // TILE
    total_tiles = (off[E - 1:E, :] + sizes[E - 1:E, :]) // TILE
    iota_nt = lax.broadcasted_iota(jnp.int32, (E, NT), 1)
    cnt = jnp.sum((tiles_before <= iota_nt).astype(jnp.int32), axis=0,
                  keepdims=True) - 1
    iota_row = lax.broadcasted_iota(jnp.int32, (1, NT), 1)
    te_ref[...] = jnp.where(iota_row < total_tiles, cnt, -1)


def _run_router(flat_x, w2, znt):
    return pl.pallas_call(
        _router_body,
        out_shape=[
            jax.ShapeDtypeStruct((TOPK, N), jnp.int32),
            jax.ShapeDtypeStruct((TOPK, N), jnp.float32),
            jax.ShapeDtypeStruct((1, NT), jnp.int32),
        ],
    )(flat_x, w2, znt)


_SROWS = NPAD // NW


def _scatter_body(pos_hbm, w_hbm, tok_out, gate_out, pos_v, w_v, dtok_v,
                  dgate_v):
    wid = lax.axis_index("s") * NC + lax.axis_index("c")
    base = wid * _SROWS
    pltpu.sync_copy(pos_hbm, pos_v)
    pltpu.sync_copy(w_hbm, w_v)
    lane = lax.iota(jnp.int32, 16)

    def scat_step(q, _):
        idx = pos_v[pl.ds(q * 16, 16)] - base
        mask = (idx >= 0) & (idx < _SROWS)
        tok = (lane + q * 16) & (N - 1)
        plsc.store_scatter(dtok_v, [idx], tok, mask=mask)
        plsc.store_scatter(dgate_v, [idx], w_v[pl.ds(q * 16, 16)], mask=mask)
        return 0

    lax.fori_loop(0, (N * TOPK) // 16, scat_step, 0)
    pltpu.sync_copy(dtok_v, tok_out.at[pl.ds(base, _SROWS)])
    pltpu.sync_copy(dgate_v, gate_out.at[pl.ds(base, _SROWS)])


def _run_scatter(pos_flat, w_flat):
    mesh = plsc.VectorSubcoreMesh(core_axis_name="c", subcore_axis_name="s")
    f = pl.kernel(
        _scatter_body,
        out_type=[
            jax.ShapeDtypeStruct((NPAD,), jnp.int32),
            jax.ShapeDtypeStruct((NPAD,), jnp.float32),
        ],
        mesh=mesh,
        compiler_params=pltpu.CompilerParams(needs_layout_passes=False),
        scratch_types=[
            pltpu.VMEM((N * TOPK,), jnp.int32),
            pltpu.VMEM((N * TOPK,), jnp.float32),
            pltpu.VMEM((_SROWS,), jnp.int32),
            pltpu.VMEM((_SROWS,), jnp.float32),
        ],
    )
    return f(pos_flat, w_flat)


_GROWS = NPAD // NW
_GCHUNK = _GROWS // 4


_GC = _GROWS // 3


def _gather_body(tok_hbm, x_hbm, out_hbm, idx_v, rows0, rows1, gsem, wsem):
    wid = lax.axis_index("s") * NC + lax.axis_index("c")
    base = wid * _GROWS
    pltpu.sync_copy(tok_hbm.at[pl.ds(base, _GROWS)], idx_v)
    bufs = (rows0, rows1, rows0)
    g = [None, None, None]
    w = [None, None, None]
    g[0] = pltpu.async_copy(x_hbm.at[idx_v.at[pl.ds(0, _GC)]], bufs[0], gsem)
    g[1] = pltpu.async_copy(x_hbm.at[idx_v.at[pl.ds(_GC, _GC)]], bufs[1], gsem)
    g[0].wait()
    w[0] = pltpu.async_copy(bufs[0], out_hbm.at[pl.ds(base, _GC)], wsem)
    g[1].wait()
    w[1] = pltpu.async_copy(bufs[1], out_hbm.at[pl.ds(base + _GC, _GC)], wsem)
    w[0].wait()
    g[2] = pltpu.async_copy(
        x_hbm.at[idx_v.at[pl.ds(2 * _GC, _GC)]], bufs[2], gsem)
    g[2].wait()
    w[2] = pltpu.async_copy(
        bufs[2], out_hbm.at[pl.ds(base + 2 * _GC, _GC)], wsem)
    w[1].wait()
    w[2].wait()


def _run_gather(row_token, flat_x):
    mesh = plsc.VectorSubcoreMesh(core_axis_name="c", subcore_axis_name="s")
    f = pl.kernel(
        _gather_body,
        out_type=jax.ShapeDtypeStruct((NPAD, D), jnp.float32),
        mesh=mesh,
        compiler_params=pltpu.CompilerParams(needs_layout_passes=False),
        scratch_types=[
            pltpu.VMEM((_GROWS,), jnp.int32),
            pltpu.VMEM((_GC, D), jnp.float32),
            pltpu.VMEM((_GC, D), jnp.float32),
            pltpu.SemaphoreType.DMA,
            pltpu.SemaphoreType.DMA,
        ],
    )
    return f(row_token, flat_x)


def _ffn_body(te_ref, tok_ref, x_ref, fc1_ref, fc2_ref, g_ref, out_ref):
    i = pl.program_id(0)
    active = te_ref[i] >= 0

    @pl.when(active)
    def _():
        tok = tok_ref[...]
        iota_n = lax.broadcasted_iota(jnp.int32, (TILE, N), 1)
        p = (iota_n == tok).astype(jnp.float32)
        xb = lax.dot_general(p, x_ref[...], (((1,), (0,)), ((), ())),
                             preferred_element_type=jnp.float32)
        h = lax.dot_general(xb, fc1_ref[0], (((1,), (1,)), ((), ())),
                            preferred_element_type=jnp.float32)
        h = jnp.maximum(h, 0.0)
        h = h * h
        y = lax.dot_general(h, fc2_ref[0], (((1,), (1,)), ((), ())),
                            preferred_element_type=jnp.float32)
        out_ref[...] = y * g_ref[...]

    @pl.when(jnp.logical_not(active))
    def _():
        out_ref[...] = jnp.zeros_like(out_ref)


def _run_ffn(te, row_token, xb, fc1_W, fc2_W, row_gate):
    grid_spec = pltpu.PrefetchScalarGridSpec(
        num_scalar_prefetch=1,
        grid=(NT,),
        in_specs=[
            pl.BlockSpec((TILE, 1), lambda i, te: (i, 0)),
            pl.BlockSpec((N, D), lambda i, te: (0, 0)),
            pl.BlockSpec((1, H, D),
                         lambda i, te: (jnp.where(te[i] < 0, E - 1, te[i]), 0, 0)),
            pl.BlockSpec((1, D, H),
                         lambda i, te: (jnp.where(te[i] < 0, E - 1, te[i]), 0, 0)),
            pl.BlockSpec((TILE, 1), lambda i, te: (i, 0)),
        ],
        out_specs=pl.BlockSpec((TILE, D), lambda i, te: (i, 0)),
    )
    return pl.pallas_call(
        _ffn_body,
        grid_spec=grid_spec,
        out_shape=jax.ShapeDtypeStruct((NPAD, D), jnp.float32),
        compiler_params=pltpu.CompilerParams(
            dimension_semantics=("arbitrary",)),
    )(te, row_token, xb, fc1_W, fc2_W, row_gate)


_TOKW = N // NW
_TCHUNK = _TOKW // 2


def _combine_body(pos_hbm, y_hbm, out_hbm, idx0_v, idx1_v, r0, r1, oa, ob,
                  gs0, gs1, wsem):
    wid = lax.axis_index("s") * NC + lax.axis_index("c")
    tw = wid * _TOKW
    pltpu.sync_copy(pos_hbm.at[pl.ds(tw, _TOKW)], idx0_v)
    pltpu.sync_copy(pos_hbm.at[pl.ds(N + tw, _TOKW)], idx1_v)
    outs = (oa, ob)
    wd = [None, None]
    for c in range(2):
        tb = tw + c * _TCHUNK
        g0 = pltpu.async_copy(
            y_hbm.at[idx0_v.at[pl.ds(c * _TCHUNK, _TCHUNK)]], r0, gs0)
        g1 = pltpu.async_copy(
            y_hbm.at[idx1_v.at[pl.ds(c * _TCHUNK, _TCHUNK)]], r1, gs1)
        g0.wait()
        g1.wait()
        o = outs[c]

        def add_step(j, _):
            for k in range(D // 16):
                o[j, pl.ds(k * 16, 16)] = (
                    r0[j, pl.ds(k * 16, 16)] + r1[j, pl.ds(k * 16, 16)])
            return 0

        lax.fori_loop(0, _TCHUNK, add_step, 0)
        wd[c] = pltpu.async_copy(o, out_hbm.at[pl.ds(tb, _TCHUNK)], wsem)
    wd[0].wait()
    wd[1].wait()


def _run_combine(pos_flat, y):
    mesh = plsc.VectorSubcoreMesh(core_axis_name="c", subcore_axis_name="s")
    f = pl.kernel(
        _combine_body,
        out_type=jax.ShapeDtypeStruct((N, D), jnp.float32),
        mesh=mesh,
        compiler_params=pltpu.CompilerParams(needs_layout_passes=False),
        scratch_types=[
            pltpu.VMEM((_TOKW,), jnp.int32),
            pltpu.VMEM((_TOKW,), jnp.int32),
            pltpu.VMEM((_TCHUNK, D), jnp.float32),
            pltpu.VMEM((_TCHUNK, D), jnp.float32),
            pltpu.VMEM((_TCHUNK, D), jnp.float32),
            pltpu.VMEM((_TCHUNK, D), jnp.float32),
            pltpu.SemaphoreType.DMA,
            pltpu.SemaphoreType.DMA,
            pltpu.SemaphoreType.DMA,
        ],
    )
    return f(pos_flat, y)


def kernel(x, route_W, noise_W, fc1_W, fc2_W):
    b, t, d = x.shape
    flat_x = x.reshape(-1, d)
    w2 = jnp.concatenate([route_W, noise_W], axis=0)
    znt = jax.random.normal(jax.random.key(1234), (N, E), jnp.float32).T

    pos, gates, te = _run_router(flat_x, w2, znt)
    pos_flat = pos.reshape(-1)
    w_flat = gates.reshape(-1)
    te_flat = te.reshape(-1)

    row_token, row_gate = _run_scatter(pos_flat, w_flat)
    y = _run_ffn(te_flat, row_token.reshape(NPAD, 1), flat_x, fc1_W, fc2_W,
                 row_gate.reshape(NPAD, 1))
    out = _run_combine(pos_flat, y)
    return out.reshape(b, t, d)
